# baseline (speedup 1.0000x reference)
import functools

import numpy as np
import jax
import jax.numpy as jnp
from jax import lax
from jax.experimental import pallas as pl
from jax.experimental.pallas import tpu as pltpu

_POS_T = 0.5
_NEG_T = 0.4
_BACKGROUND = -1.0
_IGNORE = -2.0
_VAR = (0.1, 0.1, 0.2, 0.2)

_NPAD = 104
_A = 16384


def _anchors_np(height, width):
    all_anchors = []
    for level in range(3, 8):
        stride = 2 ** level
        size = float(2 ** (level + 2))
        scales = [2.0 ** 0.0, 2.0 ** (1.0 / 3.0), 2.0 ** (2.0 / 3.0)]
        ratios = [0.5, 1.0, 2.0]
        fh = int(np.ceil(height / stride))
        fw = int(np.ceil(width / stride))
        cx = (np.arange(fw, dtype=np.float32) + 0.5) * stride
        cy = (np.arange(fh, dtype=np.float32) + 0.5) * stride
        cxg, cyg = np.meshgrid(cx, cy)
        ws, hs = [], []
        for scale in scales:
            for ratio in ratios:
                ws.append(size * scale * np.sqrt(ratio))
                hs.append(size * scale / np.sqrt(ratio))
        ws = np.asarray(ws, dtype=np.float32)
        hs = np.asarray(hs, dtype=np.float32)
        cxg = cxg[:, :, None].astype(np.float32)
        cyg = cyg[:, :, None].astype(np.float32)
        x1 = cxg - ws / 2.0
        y1 = cyg - hs / 2.0
        x2 = cxg + ws / 2.0
        y2 = cyg + hs / 2.0
        boxes = np.stack([x1, y1, x2, y2], axis=-1).reshape(-1, 4)
        all_anchors.append(boxes)
    return np.concatenate(all_anchors, axis=0)


@functools.lru_cache(maxsize=4)
def _anchor_pack(height, width):
    a = _anchors_np(height, width)
    m = a.shape[0]
    m_pad = ((m + _A - 1) // _A) * _A
    pad = np.tile(np.array([[0.0, 0.0, 16.0, 16.0]], np.float32), (m_pad - m, 1))
    a = np.concatenate([a, pad], axis=0)
    x1, y1, x2, y2 = a[:, 0], a[:, 1], a[:, 2], a[:, 3]
    w = x2 - x1
    h = y2 - y1
    area = np.clip(w, 0.0, None) * np.clip(h, 0.0, None)
    cx = (x1 + x2) * np.float32(0.5)
    cy = (y1 + y2) * np.float32(0.5)
    pack = np.zeros((16, m_pad), np.float32)
    for i, row in enumerate((x1, y1, x2, y2, area, cx, cy, w, h)):
        pack[i] = row
    return jnp.asarray(pack), m, m_pad


def _body(apack_ref, gcols_ref, grows_ref, box_ref, cls_ref):
    ap = apack_ref[...]
    g = gcols_ref[0]
    gx1 = g[:, 0:1]
    gy1 = g[:, 1:2]
    gx2 = g[:, 2:3]
    gy2 = g[:, 3:4]
    area_b = jnp.maximum(gx2 - gx1, 0.0) * jnp.maximum(gy2 - gy1, 0.0)

    ax1 = ap[0:1]
    ay1 = ap[1:2]
    ax2 = ap[2:3]
    ay2 = ap[3:4]
    area_a = ap[4:5]

    ltx = jnp.maximum(ax1, gx1)
    lty = jnp.maximum(ay1, gy1)
    rbx = jnp.minimum(ax2, gx2)
    rby = jnp.minimum(ay2, gy2)
    iw = jnp.maximum(rbx - ltx, 0.0)
    ih = jnp.maximum(rby - lty, 0.0)
    inter = iw * ih
    union = area_a + area_b - inter
    iou = inter / union

    mx = jnp.max(iou, axis=0, keepdims=True)
    rows = lax.broadcasted_iota(jnp.int32, iou.shape, 0)
    idx = jnp.argmax(iou, axis=0)[None, :]

    onehot = (rows == idx).astype(jnp.bfloat16)
    g8 = grows_ref[0]
    ghi = g8.astype(jnp.bfloat16)
    glo = (g8 - ghi.astype(jnp.float32)).astype(jnp.bfloat16)
    g16 = jnp.concatenate([ghi, glo], axis=0)
    m16 = lax.dot_general(
        g16, onehot,
        dimension_numbers=(((1,), (0,)), ((), ())),
        preferred_element_type=jnp.float32,
    )
    matched = m16[0:8] + m16[8:16]
    mx1 = matched[0:1]
    my1 = matched[1:2]
    mx2 = matched[2:3]
    my2 = matched[3:4]
    mcls = matched[4:5]

    acx = ap[5:6]
    acy = ap[6:7]
    aw = ap[7:8]
    ah = ap[8:9]
    gcx = (mx1 + mx2) * 0.5
    gcy = (my1 + my2) * 0.5
    gw = mx2 - mx1
    gh = my2 - my1

    dy = (gcy - acy) / ah / _VAR[0]
    dx = (gcx - acx) / aw / _VAR[1]
    dh = jnp.log(jnp.maximum(gh, 1e-8) / ah) / _VAR[2]
    dw = jnp.log(jnp.maximum(gw, 1e-8) / aw) / _VAR[3]

    positive = mx >= _POS_T
    ignore = jnp.logical_and(mx >= _NEG_T, mx < _POS_T)
    cls_t = jnp.where(positive, mcls, _BACKGROUND)
    cls_t = jnp.where(ignore, _IGNORE, cls_t)


    box_ref[0] = jnp.concatenate([dy, dx, dh, dw], axis=0)
    cls_ref[0] = cls_t


def kernel(images, gt_boxes, gt_classes):
    B = images.shape[0]
    H, W = images.shape[1], images.shape[2]
    apack, m, m_pad = _anchor_pack(H, W)
    n = gt_boxes.shape[1]

    gcols = jnp.zeros((B, _NPAD, 8), jnp.float32)
    gcols = gcols.at[:, :n, 0:4].set(gt_boxes)
    gcols = gcols.at[:, :n, 4].set(gt_classes)
    grows = jnp.transpose(gcols, (0, 2, 1))

    nb = m_pad // _A
    box_p, cls_p = pl.pallas_call(
        _body,
        grid=(B, nb),
        in_specs=[
            pl.BlockSpec((16, _A), lambda b, j: (0, j)),
            pl.BlockSpec((1, _NPAD, 8), lambda b, j: (b, 0, 0)),
            pl.BlockSpec((1, 8, _NPAD), lambda b, j: (b, 0, 0)),
        ],
        out_specs=[
            pl.BlockSpec((1, 4, _A), lambda b, j: (b, 0, j)),
            pl.BlockSpec((1, 1, _A), lambda b, j: (b, 0, j)),
        ],
        out_shape=[
            jax.ShapeDtypeStruct((B, 4, m_pad), jnp.float32),
            jax.ShapeDtypeStruct((B, 1, m_pad), jnp.float32),
        ],
        compiler_params=pltpu.CompilerParams(
            dimension_semantics=("parallel", "parallel"),
        ),
    )(apack, gcols, grows)

    box = jnp.transpose(box_p[:, :, :m], (0, 2, 1))
    cls = cls_p[:, 0, :m]
    return box, cls


# speedup vs baseline: 1.0288x; 1.0288x over previous
import functools

import numpy as np
import jax
import jax.numpy as jnp
from jax import lax
from jax.experimental import pallas as pl
from jax.experimental.pallas import tpu as pltpu

_POS_T = 0.5
_NEG_T = 0.4
_BACKGROUND = -1.0
_IGNORE = -2.0
_VAR = (0.1, 0.1, 0.2, 0.2)

_NPAD = 104
_A = 16384


def _anchors_np(height, width):
    all_anchors = []
    for level in range(3, 8):
        stride = 2 ** level
        size = float(2 ** (level + 2))
        scales = [2.0 ** 0.0, 2.0 ** (1.0 / 3.0), 2.0 ** (2.0 / 3.0)]
        ratios = [0.5, 1.0, 2.0]
        fh = int(np.ceil(height / stride))
        fw = int(np.ceil(width / stride))
        cx = (np.arange(fw, dtype=np.float32) + 0.5) * stride
        cy = (np.arange(fh, dtype=np.float32) + 0.5) * stride
        cxg, cyg = np.meshgrid(cx, cy)
        ws, hs = [], []
        for scale in scales:
            for ratio in ratios:
                ws.append(size * scale * np.sqrt(ratio))
                hs.append(size * scale / np.sqrt(ratio))
        ws = np.asarray(ws, dtype=np.float32)
        hs = np.asarray(hs, dtype=np.float32)
        cxg = cxg[:, :, None].astype(np.float32)
        cyg = cyg[:, :, None].astype(np.float32)
        x1 = cxg - ws / 2.0
        y1 = cyg - hs / 2.0
        x2 = cxg + ws / 2.0
        y2 = cyg + hs / 2.0
        boxes = np.stack([x1, y1, x2, y2], axis=-1).reshape(-1, 4)
        all_anchors.append(boxes)
    return np.concatenate(all_anchors, axis=0)


@functools.lru_cache(maxsize=4)
def _anchor_pack(height, width):
    a = _anchors_np(height, width)
    m = a.shape[0]
    m_pad = ((m + _A - 1) // _A) * _A
    pad = np.tile(np.array([[0.0, 0.0, 16.0, 16.0]], np.float32), (m_pad - m, 1))
    a = np.concatenate([a, pad], axis=0)
    x1, y1, x2, y2 = a[:, 0], a[:, 1], a[:, 2], a[:, 3]
    w = x2 - x1
    h = y2 - y1
    area = np.clip(w, 0.0, None) * np.clip(h, 0.0, None)
    cx = (x1 + x2) * np.float32(0.5)
    cy = (y1 + y2) * np.float32(0.5)
    pack = np.zeros((16, m_pad), np.float32)
    for i, row in enumerate((x1, y1, x2, y2, area, cx, cy, w, h)):
        pack[i] = row
    return jnp.asarray(pack), m, m_pad


def _body(apack_ref, gcols_ref, grows_ref, box_ref, cls_ref):
    ap = apack_ref[...]
    g = gcols_ref[0]
    gx1 = g[:, 0:1]
    gy1 = g[:, 1:2]
    gx2 = g[:, 2:3]
    gy2 = g[:, 3:4]
    area_b = jnp.maximum(gx2 - gx1, 0.0) * jnp.maximum(gy2 - gy1, 0.0)

    ax1 = ap[0:1]
    ay1 = ap[1:2]
    ax2 = ap[2:3]
    ay2 = ap[3:4]
    area_a = ap[4:5]

    ltx = jnp.maximum(ax1, gx1)
    lty = jnp.maximum(ay1, gy1)
    rbx = jnp.minimum(ax2, gx2)
    rby = jnp.minimum(ay2, gy2)
    iw = jnp.maximum(rbx - ltx, 0.0)
    ih = jnp.maximum(rby - lty, 0.0)
    inter = iw * ih
    union = area_a + area_b - inter
    iou = inter / union

    mx = jnp.max(iou, axis=0, keepdims=True)
    rows = lax.broadcasted_iota(jnp.int32, iou.shape, 0)
    idx = jnp.min(jnp.where(iou == mx, rows, _NPAD), axis=0, keepdims=True)

    onehot = (rows == idx).astype(jnp.bfloat16)
    g8 = grows_ref[0]
    ghi = g8.astype(jnp.bfloat16)
    glo = (g8 - ghi.astype(jnp.float32)).astype(jnp.bfloat16)
    g16 = jnp.concatenate([ghi, glo], axis=0)
    m16 = lax.dot_general(
        g16, onehot,
        dimension_numbers=(((1,), (0,)), ((), ())),
        preferred_element_type=jnp.float32,
    )
    matched = m16[0:8] + m16[8:16]
    mx1 = matched[0:1]
    my1 = matched[1:2]
    mx2 = matched[2:3]
    my2 = matched[3:4]
    mcls = matched[4:5]

    acx = ap[5:6]
    acy = ap[6:7]
    aw = ap[7:8]
    ah = ap[8:9]
    gcx = (mx1 + mx2) * 0.5
    gcy = (my1 + my2) * 0.5
    gw = mx2 - mx1
    gh = my2 - my1

    dy = (gcy - acy) / ah / _VAR[0]
    dx = (gcx - acx) / aw / _VAR[1]
    dh = jnp.log(jnp.maximum(gh, 1e-8) / ah) / _VAR[2]
    dw = jnp.log(jnp.maximum(gw, 1e-8) / aw) / _VAR[3]

    positive = mx >= _POS_T
    ignore = jnp.logical_and(mx >= _NEG_T, mx < _POS_T)
    cls_t = jnp.where(positive, mcls, _BACKGROUND)
    cls_t = jnp.where(ignore, _IGNORE, cls_t)


    box_ref[0] = jnp.concatenate([dy, dx, dh, dw], axis=0)
    cls_ref[0] = cls_t


def kernel(images, gt_boxes, gt_classes):
    B = images.shape[0]
    H, W = images.shape[1], images.shape[2]
    apack, m, m_pad = _anchor_pack(H, W)
    n = gt_boxes.shape[1]

    gcols = jnp.zeros((B, _NPAD, 8), jnp.float32)
    gcols = gcols.at[:, :n, 0:4].set(gt_boxes)
    gcols = gcols.at[:, :n, 4].set(gt_classes)
    grows = jnp.transpose(gcols, (0, 2, 1))

    nb = m_pad // _A
    box_p, cls_p = pl.pallas_call(
        _body,
        grid=(B, nb),
        in_specs=[
            pl.BlockSpec((16, _A), lambda b, j: (0, j)),
            pl.BlockSpec((1, _NPAD, 8), lambda b, j: (b, 0, 0)),
            pl.BlockSpec((1, 8, _NPAD), lambda b, j: (b, 0, 0)),
        ],
        out_specs=[
            pl.BlockSpec((1, 4, _A), lambda b, j: (b, 0, j)),
            pl.BlockSpec((1, 1, _A), lambda b, j: (b, 0, j)),
        ],
        out_shape=[
            jax.ShapeDtypeStruct((B, 4, m_pad), jnp.float32),
            jax.ShapeDtypeStruct((B, 1, m_pad), jnp.float32),
        ],
        compiler_params=pltpu.CompilerParams(
            dimension_semantics=("parallel", "parallel"),
        ),
    )(apack, gcols, grows)

    box = jnp.transpose(box_p[:, :, :m], (0, 2, 1))
    cls = cls_p[:, 0, :m]
    return box, cls


# speedup vs baseline: 1.1048x; 1.0738x over previous
import functools

import numpy as np
import jax
import jax.numpy as jnp
from jax import lax
from jax.experimental import pallas as pl
from jax.experimental.pallas import tpu as pltpu

_POS_T = 0.5
_NEG_T = 0.4
_BACKGROUND = -1.0
_IGNORE = -2.0
_VAR = (0.1, 0.1, 0.2, 0.2)

_NPAD = 104
_A = 16384


def _anchors_np(height, width):
    all_anchors = []
    for level in range(3, 8):
        stride = 2 ** level
        size = float(2 ** (level + 2))
        scales = [2.0 ** 0.0, 2.0 ** (1.0 / 3.0), 2.0 ** (2.0 / 3.0)]
        ratios = [0.5, 1.0, 2.0]
        fh = int(np.ceil(height / stride))
        fw = int(np.ceil(width / stride))
        cx = (np.arange(fw, dtype=np.float32) + 0.5) * stride
        cy = (np.arange(fh, dtype=np.float32) + 0.5) * stride
        cxg, cyg = np.meshgrid(cx, cy)
        ws, hs = [], []
        for scale in scales:
            for ratio in ratios:
                ws.append(size * scale * np.sqrt(ratio))
                hs.append(size * scale / np.sqrt(ratio))
        ws = np.asarray(ws, dtype=np.float32)
        hs = np.asarray(hs, dtype=np.float32)
        cxg = cxg[:, :, None].astype(np.float32)
        cyg = cyg[:, :, None].astype(np.float32)
        x1 = cxg - ws / 2.0
        y1 = cyg - hs / 2.0
        x2 = cxg + ws / 2.0
        y2 = cyg + hs / 2.0
        boxes = np.stack([x1, y1, x2, y2], axis=-1).reshape(-1, 4)
        all_anchors.append(boxes)
    return np.concatenate(all_anchors, axis=0)


@functools.lru_cache(maxsize=4)
def _anchor_pack(height, width):
    a = _anchors_np(height, width)
    m = a.shape[0]
    m_pad = ((m + _A - 1) // _A) * _A
    pad = np.tile(np.array([[0.0, 0.0, 16.0, 16.0]], np.float32), (m_pad - m, 1))
    a = np.concatenate([a, pad], axis=0)
    x1, y1, x2, y2 = a[:, 0], a[:, 1], a[:, 2], a[:, 3]
    w = x2 - x1
    h = y2 - y1
    area = np.clip(w, 0.0, None) * np.clip(h, 0.0, None)
    cx = (x1 + x2) * np.float32(0.5)
    cy = (y1 + y2) * np.float32(0.5)
    pack = np.zeros((16, m_pad), np.float32)
    for i, row in enumerate((x1, y1, x2, y2, area, cx, cy, w, h)):
        pack[i] = row
    return jnp.asarray(pack), m, m_pad


def _body(apack_ref, gcols_ref, grows_ref, box_ref, cls_ref):
    ap = apack_ref[...]
    g = gcols_ref[0]
    gx1 = g[:, 0:1]
    gy1 = g[:, 1:2]
    gx2 = g[:, 2:3]
    gy2 = g[:, 3:4]
    area_b = jnp.maximum(gx2 - gx1, 0.0) * jnp.maximum(gy2 - gy1, 0.0)

    ax1 = ap[0:1]
    ay1 = ap[1:2]
    ax2 = ap[2:3]
    ay2 = ap[3:4]
    area_a = ap[4:5]

    ltx = jnp.maximum(ax1, gx1)
    lty = jnp.maximum(ay1, gy1)
    rbx = jnp.minimum(ax2, gx2)
    rby = jnp.minimum(ay2, gy2)
    iw = jnp.maximum(rbx - ltx, 0.0)
    ih = jnp.maximum(rby - lty, 0.0)
    inter = iw * ih
    union = area_a + area_b - inter
    iou = inter / union

    mx = jnp.max(iou, axis=0, keepdims=True)
    rows = lax.broadcasted_iota(jnp.int32, iou.shape, 0)
    idx = jnp.min(jnp.where(iou == mx, rows, _NPAD), axis=0, keepdims=True)

    onehot = (rows == idx).astype(jnp.bfloat16)
    g8 = grows_ref[0]
    ghi = g8.astype(jnp.bfloat16)
    glo = (g8 - ghi.astype(jnp.float32)).astype(jnp.bfloat16)
    g16 = jnp.concatenate([ghi, glo], axis=0)
    m16 = lax.dot_general(
        g16, onehot,
        dimension_numbers=(((1,), (0,)), ((), ())),
        preferred_element_type=jnp.float32,
    )
    matched = m16[0:8] + m16[8:16]
    mx1 = matched[0:1]
    my1 = matched[1:2]
    mx2 = matched[2:3]
    my2 = matched[3:4]
    mcls = matched[4:5]

    acx = ap[5:6]
    acy = ap[6:7]
    aw = ap[7:8]
    ah = ap[8:9]
    gcx = (mx1 + mx2) * 0.5
    gcy = (my1 + my2) * 0.5
    gw = mx2 - mx1
    gh = my2 - my1

    dy = (gcy - acy) / ah * (1.0 / _VAR[0])
    dx = (gcx - acx) / aw * (1.0 / _VAR[1])
    dh = jnp.log(jnp.maximum(gh, 1e-8) / ah) * (1.0 / _VAR[2])
    dw = jnp.log(jnp.maximum(gw, 1e-8) / aw) * (1.0 / _VAR[3])

    positive = mx >= _POS_T
    ignore = jnp.logical_and(mx >= _NEG_T, mx < _POS_T)
    cls_t = jnp.where(positive, mcls, _BACKGROUND)
    cls_t = jnp.where(ignore, _IGNORE, cls_t)


    box_ref[0] = jnp.concatenate([dy, dx, dh, dw], axis=0)
    cls_ref[0] = cls_t


def kernel(images, gt_boxes, gt_classes):
    B = images.shape[0]
    H, W = images.shape[1], images.shape[2]
    apack, m, m_pad = _anchor_pack(H, W)
    n = gt_boxes.shape[1]

    gcat = jnp.concatenate([gt_boxes, gt_classes[:, :, None]], axis=-1)
    gcols = jnp.pad(gcat, ((0, 0), (0, _NPAD - n), (0, 3)))
    grows = jnp.transpose(gcols, (0, 2, 1))

    nb = m_pad // _A
    box_p, cls_p = pl.pallas_call(
        _body,
        grid=(B, nb),
        in_specs=[
            pl.BlockSpec((16, _A), lambda b, j: (0, j)),
            pl.BlockSpec((1, _NPAD, 8), lambda b, j: (b, 0, 0)),
            pl.BlockSpec((1, 8, _NPAD), lambda b, j: (b, 0, 0)),
        ],
        out_specs=[
            pl.BlockSpec((1, 4, _A), lambda b, j: (b, 0, j)),
            pl.BlockSpec((1, 1, _A), lambda b, j: (b, 0, j)),
        ],
        out_shape=[
            jax.ShapeDtypeStruct((B, 4, m_pad), jnp.float32),
            jax.ShapeDtypeStruct((B, 1, m_pad), jnp.float32),
        ],
        compiler_params=pltpu.CompilerParams(
            dimension_semantics=("parallel", "parallel"),
        ),
    )(apack, gcols, grows)

    box = jnp.transpose(box_p[:, :, :m], (0, 2, 1))
    cls = cls_p[:, 0, :m]
    return box, cls


# speedup vs baseline: 1.1125x; 1.0070x over previous
import functools

import numpy as np
import jax
import jax.numpy as jnp
from jax import lax
from jax.experimental import pallas as pl
from jax.experimental.pallas import tpu as pltpu

_POS_T = 0.5
_NEG_T = 0.4
_BACKGROUND = -1.0
_IGNORE = -2.0
_VAR = (0.1, 0.1, 0.2, 0.2)

_NPAD = 104
_A = 24576


def _anchors_np(height, width):
    all_anchors = []
    for level in range(3, 8):
        stride = 2 ** level
        size = float(2 ** (level + 2))
        scales = [2.0 ** 0.0, 2.0 ** (1.0 / 3.0), 2.0 ** (2.0 / 3.0)]
        ratios = [0.5, 1.0, 2.0]
        fh = int(np.ceil(height / stride))
        fw = int(np.ceil(width / stride))
        cx = (np.arange(fw, dtype=np.float32) + 0.5) * stride
        cy = (np.arange(fh, dtype=np.float32) + 0.5) * stride
        cxg, cyg = np.meshgrid(cx, cy)
        ws, hs = [], []
        for scale in scales:
            for ratio in ratios:
                ws.append(size * scale * np.sqrt(ratio))
                hs.append(size * scale / np.sqrt(ratio))
        ws = np.asarray(ws, dtype=np.float32)
        hs = np.asarray(hs, dtype=np.float32)
        cxg = cxg[:, :, None].astype(np.float32)
        cyg = cyg[:, :, None].astype(np.float32)
        x1 = cxg - ws / 2.0
        y1 = cyg - hs / 2.0
        x2 = cxg + ws / 2.0
        y2 = cyg + hs / 2.0
        boxes = np.stack([x1, y1, x2, y2], axis=-1).reshape(-1, 4)
        all_anchors.append(boxes)
    return np.concatenate(all_anchors, axis=0)


@functools.lru_cache(maxsize=4)
def _anchor_pack(height, width):
    a = _anchors_np(height, width)
    m = a.shape[0]
    m_pad = ((m + _A - 1) // _A) * _A
    pad = np.tile(np.array([[0.0, 0.0, 16.0, 16.0]], np.float32), (m_pad - m, 1))
    a = np.concatenate([a, pad], axis=0)
    x1, y1, x2, y2 = a[:, 0], a[:, 1], a[:, 2], a[:, 3]
    w = x2 - x1
    h = y2 - y1
    area = np.clip(w, 0.0, None) * np.clip(h, 0.0, None)
    cx = (x1 + x2) * np.float32(0.5)
    cy = (y1 + y2) * np.float32(0.5)
    pack = np.zeros((16, m_pad), np.float32)
    for i, row in enumerate((x1, y1, x2, y2, area, cx, cy, w, h)):
        pack[i] = row
    return jnp.asarray(pack), m, m_pad


def _body(apack_ref, gcols_ref, grows_ref, box_ref, cls_ref):
    ap = apack_ref[...]
    g = gcols_ref[0]
    gx1 = g[:, 0:1]
    gy1 = g[:, 1:2]
    gx2 = g[:, 2:3]
    gy2 = g[:, 3:4]
    area_b = jnp.maximum(gx2 - gx1, 0.0) * jnp.maximum(gy2 - gy1, 0.0)

    ax1 = ap[0:1]
    ay1 = ap[1:2]
    ax2 = ap[2:3]
    ay2 = ap[3:4]
    area_a = ap[4:5]

    ltx = jnp.maximum(ax1, gx1)
    lty = jnp.maximum(ay1, gy1)
    rbx = jnp.minimum(ax2, gx2)
    rby = jnp.minimum(ay2, gy2)
    iw = jnp.maximum(rbx - ltx, 0.0)
    ih = jnp.maximum(rby - lty, 0.0)
    inter = iw * ih
    union = area_a + area_b - inter
    iou = inter / union

    mx = jnp.max(iou, axis=0, keepdims=True)
    rows = lax.broadcasted_iota(jnp.int32, iou.shape, 0)
    idx = jnp.min(jnp.where(iou == mx, rows, _NPAD), axis=0, keepdims=True)

    onehot = (rows == idx).astype(jnp.bfloat16)
    g8 = grows_ref[0]
    ghi = g8.astype(jnp.bfloat16)
    glo = (g8 - ghi.astype(jnp.float32)).astype(jnp.bfloat16)
    g16 = jnp.concatenate([ghi, glo], axis=0)
    m16 = lax.dot_general(
        g16, onehot,
        dimension_numbers=(((1,), (0,)), ((), ())),
        preferred_element_type=jnp.float32,
    )
    matched = m16[0:8] + m16[8:16]
    mx1 = matched[0:1]
    my1 = matched[1:2]
    mx2 = matched[2:3]
    my2 = matched[3:4]
    mcls = matched[4:5]

    acx = ap[5:6]
    acy = ap[6:7]
    aw = ap[7:8]
    ah = ap[8:9]
    gcx = (mx1 + mx2) * 0.5
    gcy = (my1 + my2) * 0.5
    gw = mx2 - mx1
    gh = my2 - my1

    dy = (gcy - acy) / ah * (1.0 / _VAR[0])
    dx = (gcx - acx) / aw * (1.0 / _VAR[1])
    dh = jnp.log(jnp.maximum(gh, 1e-8) / ah) * (1.0 / _VAR[2])
    dw = jnp.log(jnp.maximum(gw, 1e-8) / aw) * (1.0 / _VAR[3])

    positive = mx >= _POS_T
    ignore = jnp.logical_and(mx >= _NEG_T, mx < _POS_T)
    cls_t = jnp.where(positive, mcls, _BACKGROUND)
    cls_t = jnp.where(ignore, _IGNORE, cls_t)


    box_ref[0] = jnp.concatenate([dy, dx, dh, dw], axis=0)
    cls_ref[0] = cls_t


def kernel(images, gt_boxes, gt_classes):
    B = images.shape[0]
    H, W = images.shape[1], images.shape[2]
    apack, m, m_pad = _anchor_pack(H, W)
    n = gt_boxes.shape[1]

    gcat = jnp.concatenate([gt_boxes, gt_classes[:, :, None]], axis=-1)
    gcols = jnp.pad(gcat, ((0, 0), (0, _NPAD - n), (0, 3)))
    grows = jnp.transpose(gcols, (0, 2, 1))

    nb = m_pad // _A
    box_p, cls_p = pl.pallas_call(
        _body,
        grid=(B, nb),
        in_specs=[
            pl.BlockSpec((16, _A), lambda b, j: (0, j)),
            pl.BlockSpec((1, _NPAD, 8), lambda b, j: (b, 0, 0)),
            pl.BlockSpec((1, 8, _NPAD), lambda b, j: (b, 0, 0)),
        ],
        out_specs=[
            pl.BlockSpec((1, 4, _A), lambda b, j: (b, 0, j)),
            pl.BlockSpec((1, 1, _A), lambda b, j: (b, 0, j)),
        ],
        out_shape=[
            jax.ShapeDtypeStruct((B, 4, m_pad), jnp.float32),
            jax.ShapeDtypeStruct((B, 1, m_pad), jnp.float32),
        ],
        compiler_params=pltpu.CompilerParams(
            dimension_semantics=("parallel", "parallel"),
        ),
    )(apack, gcols, grows)

    box = jnp.transpose(box_p[:, :, :m], (0, 2, 1))
    cls = cls_p[:, 0, :m]
    return box, cls
